# baseline (speedup 1.0000x reference)
import dataclasses
import functools

import jax
import jax.numpy as jnp
from jax import lax
from jax.experimental import pallas as pl
from jax.experimental.pallas import tpu as pltpu
from jax.experimental.pallas import tpu_sc as plsc

N = 10000
E = 160000
E2 = 2 * E
D = 128
EH = 64

_NC = 2
_NS = 16
_NW = _NC * _NS
_CH = 128


def _vector_mesh():
    return plsc.VectorSubcoreMesh(core_axis_name="c", subcore_axis_name="s")


def _sc_params(tc_tiling=None):
    cp = pltpu.CompilerParams()
    fields = pltpu.CompilerParams.__dataclass_fields__
    if "needs_layout_passes" in fields:
        cp = dataclasses.replace(cp, needs_layout_passes=False)
    if tc_tiling is not None and "use_tc_tiling_on_sc" in fields:
        cp = dataclasses.replace(cp, use_tc_tiling_on_sc=tc_tiling)
    return cp


_NPAD = 10240
_HD = D // 2
_AGG_RT = 160
_AGG_NCH = E2 // _CH


def _sc_agg(x2, srcr, dstr, zrows, zdeg):
    nrow_t = _NPAD // _NS

    @functools.partial(
        pl.kernel,
        out_type=(
            jax.ShapeDtypeStruct((_NC, N, _HD), jnp.float32),
            jax.ShapeDtypeStruct((_NPAD,), jnp.float32),
        ),
        mesh=_vector_mesh(),
        compiler_params=_sc_params(tc_tiling=False),
        scratch_types=[
            pltpu.VMEM((_AGG_RT, _CH), jnp.int32),
            pltpu.VMEM((_AGG_RT, _CH), jnp.int32),
            pltpu.VMEM((_CH, _HD), jnp.float32),
            pltpu.VMEM((_CH, _HD), jnp.float32),
            pltpu.VMEM((_NPAD,), jnp.float32),
            pltpu.VMEM((nrow_t,), jnp.float32),
            pltpu.VMEM((nrow_t,), jnp.float32),
            pltpu.VMEM_SHARED((_NPAD, _HD), jnp.float32),
            pltpu.VMEM_SHARED((_NS, _NPAD), jnp.float32),
            pltpu.SemaphoreType.DMA,
            pltpu.SemaphoreType.DMA,
        ],
    )
    def k(x_hbm, src_hbm, dst_hbm, zr_hbm, zd_hbm, part_hbm, degn_hbm,
          idx_s, idx_d, rows0, rows1, deg_v, red0, red1, acc_sh, degst_sh,
          sem0, sem1):
        cid = lax.axis_index("c")
        sid = lax.axis_index("s")
        r0 = sid * nrow_t
        base = sid * _AGG_RT
        pltpu.sync_copy(zd_hbm, deg_v)
        pltpu.sync_copy(zr_hbm, acc_sh.at[pl.ds(r0, nrow_t)])
        pltpu.sync_copy(src_hbm.at[pl.ds(base, _AGG_RT)], idx_s)
        pltpu.sync_copy(dst_hbm.at[pl.ds(base, _AGG_RT)], idx_d)
        plsc.subcore_barrier()

        ones16 = jnp.full((16,), 1.0, jnp.float32)

        def hist(i):
            @pl.when(cid == 0)
            def _():
                @pl.loop(0, _CH, step=16)
                def _(j):
                    plsc.addupdate_scatter(deg_v, [idx_d[i, pl.ds(j, 16)]],
                                           ones16)

        def start_gather(i, buf, sem):
            pltpu.async_copy(x_hbm.at[cid].at[idx_s.at[i]], buf, sem)

        def wait_gather(buf, sem):
            pltpu.make_async_copy(x_hbm.at[cid].at[idx_s.at[0]], buf,
                                  sem).wait()

        def scatter(i, buf):
            pltpu.sync_copy(buf, acc_sh.at[idx_d.at[i]], add=True)

        start_gather(0, rows0, sem0)

        @pl.loop(0, _AGG_RT // 2)
        def _(t):
            i0 = 2 * t
            i1 = i0 + 1

            @pl.when(base + i1 < _AGG_NCH)
            def _():
                start_gather(i1, rows1, sem1)

            @pl.when(base + i0 < _AGG_NCH)
            def _():
                hist(i0)
                wait_gather(rows0, sem0)
                scatter(i0, rows0)

            @pl.when(jnp.logical_and(i0 + 2 < _AGG_RT,
                                     base + i0 + 2 < _AGG_NCH))
            def _():
                start_gather(i0 + 2, rows0, sem0)

            @pl.when(base + i1 < _AGG_NCH)
            def _():
                hist(i1)
                wait_gather(rows1, sem1)
                scatter(i1, rows1)

        @pl.when(cid == 0)
        def _():
            pltpu.sync_copy(deg_v, degst_sh.at[sid])

        plsc.subcore_barrier()

        @pl.when(cid == 0)
        def _():
            pltpu.sync_copy(degst_sh.at[0, pl.ds(r0, nrow_t)], red0)
            for t in range(1, _NS):
                pltpu.sync_copy(degst_sh.at[t, pl.ds(r0, nrow_t)], red1)

                @pl.loop(0, nrow_t, step=16)
                def _(j):
                    red0[pl.ds(j, 16)] = red0[pl.ds(j, 16)] + red1[pl.ds(j, 16)]

            pltpu.sync_copy(red0, degn_hbm.at[pl.ds(r0, nrow_t)])

        @pl.when(sid < _NS - 1)
        def _():
            pltpu.sync_copy(acc_sh.at[pl.ds(r0, nrow_t)],
                            part_hbm.at[cid, pl.ds(r0, nrow_t)])

        @pl.when(sid == _NS - 1)
        def _():
            pltpu.sync_copy(acc_sh.at[pl.ds(r0, N - (_NS - 1) * nrow_t)],
                            part_hbm.at[cid, pl.ds(r0, N - (_NS - 1) * nrow_t)])

    return k(x2, srcr, dstr, zrows, zdeg)


_RO_RT = 40
_RO_NCH = E // _CH
_EPAD = _RO_RT * _NW * _CH


def _sc_readout_gather(ps, pd, srcr, dstr):
    @functools.partial(
        pl.kernel,
        out_type=jax.ShapeDtypeStruct((_EPAD, D), jnp.float32),
        mesh=_vector_mesh(),
        compiler_params=_sc_params(tc_tiling=False),
        scratch_types=[
            pltpu.VMEM((_RO_RT, _CH), jnp.int32),
            pltpu.VMEM((_RO_RT, _CH), jnp.int32),
            pltpu.VMEM((_CH, EH), jnp.float32),
            pltpu.VMEM((_CH, EH), jnp.float32),
            pltpu.VMEM((_CH, EH), jnp.float32),
            pltpu.VMEM((_CH, EH), jnp.float32),
            pltpu.VMEM((_CH, D), jnp.float32),
            pltpu.VMEM((_CH, D), jnp.float32),
            pltpu.SemaphoreType.DMA,
            pltpu.SemaphoreType.DMA,
            pltpu.SemaphoreType.DMA,
            pltpu.SemaphoreType.DMA,
            pltpu.SemaphoreType.DMA,
            pltpu.SemaphoreType.DMA,
        ],
    )
    def k(ps_hbm, pd_hbm, s_hbm, d_hbm, z_hbm, idx_s, idx_d,
          rsA, rdA, rsB, rdB, rzA, rzB, gsA, gdA, goA, gsB, gdB, goB):
        cid = lax.axis_index("c")
        sid = lax.axis_index("s")
        wid = cid * _NS + sid
        base = wid * _RO_RT
        pltpu.sync_copy(s_hbm.at[pl.ds(base, _RO_RT)], idx_s)
        pltpu.sync_copy(d_hbm.at[pl.ds(base, _RO_RT)], idx_d)

        def start_gathers(i, rs, rd, ss, sd):
            pltpu.async_copy(ps_hbm.at[idx_s.at[i]], rs, ss)
            pltpu.async_copy(pd_hbm.at[idx_d.at[i]], rd, sd)

        def wait_gathers(rs, rd, ss, sd):
            pltpu.make_async_copy(ps_hbm.at[idx_s.at[0]], rs, ss).wait()
            pltpu.make_async_copy(pd_hbm.at[idx_d.at[0]], rd, sd).wait()

        def add_rows(rs, rd, rz):
            @pl.loop(0, _CH)
            def _(r):
                for j in range(EH // 16):
                    sl = pl.ds(j * 16, 16)
                    rz[r, sl] = rs[r, sl] + rd[r, sl]

        def start_store(i, rz, so):
            pltpu.async_copy(rz, z_hbm.at[pl.ds((base + i) * _CH, _CH)], so)

        def drain_store(rz, so):
            pltpu.make_async_copy(rz, z_hbm.at[pl.ds(0, _CH)], so).wait()

        start_gathers(0, rsA, rdA, gsA, gdA)

        @pl.loop(0, _RO_RT // 2)
        def _(t):
            i0 = 2 * t
            i1 = i0 + 1
            i2 = i0 + 2

            @pl.when(base + i1 < _RO_NCH)
            def _():
                @pl.when(t > 0)
                def _():
                    drain_store(rzB, goB)

                start_gathers(i1, rsB, rdB, gsB, gdB)

            @pl.when(base + i0 < _RO_NCH)
            def _():
                wait_gathers(rsA, rdA, gsA, gdA)
                add_rows(rsA, rdA, rzA)
                start_store(i0, rzA, goA)

            @pl.when(jnp.logical_and(i2 < _RO_RT, base + i2 < _RO_NCH))
            def _():
                drain_store(rzA, goA)
                start_gathers(i2, rsA, rdA, gsA, gdA)

            @pl.when(base + i1 < _RO_NCH)
            def _():
                wait_gathers(rsB, rdB, gsB, gdB)
                add_rows(rsB, rdB, rzB)
                start_store(i1, rzB, goB)

        @pl.when(base < _RO_NCH)
        def _():
            drain_store(rzA, goA)

        @pl.when(base + 1 < _RO_NCH)
        def _():
            drain_store(rzB, goB)

    return k(ps, pd, srcr, dstr)


def _tc_matmul_bias(x, w, brow):
    R = 2000

    def body(x_ref, w_ref, b_ref, o_ref):
        y = jnp.dot(x_ref[...], w_ref[...],
                    preferred_element_type=jnp.float32) + b_ref[...]
        o_ref[0] = y[:, :_HD]
        o_ref[1] = y[:, _HD:]

    return pl.pallas_call(
        body,
        grid=(N // R,),
        in_specs=[
            pl.BlockSpec((R, D), lambda i: (i, 0)),
            pl.BlockSpec((D, D), lambda i: (0, 0)),
            pl.BlockSpec((1, D), lambda i: (0, 0)),
        ],
        out_specs=pl.BlockSpec((2, R, _HD), lambda i: (0, i, 0)),
        out_shape=jax.ShapeDtypeStruct((2, N, _HD), jnp.float32),
    )(x, w, brow)


def _tc_norm_mm(part, degn, brow, w, relu, split_out):
    R = 2000

    def body(p_ref, d_ref, b_ref, w_ref, *o_refs):
        agg = jnp.concatenate([p_ref[0], p_ref[1]], axis=1)
        deg = jnp.maximum(d_ref[...], 1.0)
        h = agg / deg + b_ref[...]
        if relu:
            h = jnp.maximum(h, 0.0)
        ya = jnp.dot(h, w_ref[:, :_HD], preferred_element_type=jnp.float32)
        yb = jnp.dot(h, w_ref[:, _HD:], preferred_element_type=jnp.float32)
        if split_out:
            o_refs[0][...] = ya
            o_refs[1][...] = yb
        else:
            o_refs[0][0] = ya
            o_refs[0][1] = yb

    if split_out:
        out_specs = [pl.BlockSpec((R, _HD), lambda i: (i, 0)),
                     pl.BlockSpec((R, _HD), lambda i: (i, 0))]
        out_shape = [jax.ShapeDtypeStruct((N, _HD), jnp.float32),
                     jax.ShapeDtypeStruct((N, _HD), jnp.float32)]
    else:
        out_specs = pl.BlockSpec((2, R, _HD), lambda i: (0, i, 0))
        out_shape = jax.ShapeDtypeStruct((2, N, _HD), jnp.float32)

    return pl.pallas_call(
        body,
        grid=(N // R,),
        in_specs=[
            pl.BlockSpec((2, R, _HD), lambda i: (0, i, 0)),
            pl.BlockSpec((R, 1), lambda i: (i, 0)),
            pl.BlockSpec((1, D), lambda i: (0, 0)),
            pl.BlockSpec((D, D), lambda i: (0, 0)),
        ],
        out_specs=out_specs,
        out_shape=out_shape,
    )(part, degn, brow, w)


def _tc_readout(zp, angp, gtp, actp, wa, wg, wc, berow, worow, bo11):
    NCJ = 64
    RZ = NCJ * _CH

    def body(z_ref, a_ref, g_ref, c_ref, wa_ref, wg_ref, wc_ref, be_ref,
             wo_ref, bo_ref, o_ref):
        wa = wa_ref[...]
        wg = wg_ref[...]
        wc = wc_ref[...]
        be = be_ref[...]
        wo = wo_ref[...]
        bo = bo_ref[0, 0]
        for j in range(NCJ):
            zj = jnp.transpose(z_ref[pl.ds(j * _CH, _CH), :EH])
            zz = (zj + a_ref[j:j + 1, :] * wa + g_ref[j:j + 1, :] * wg
                  + c_ref[j:j + 1, :] * wc + be)
            h = jnp.maximum(zz, 0.0)
            o_ref[j:j + 1, :] = (jnp.sum(h * wo, axis=0, keepdims=True) + bo)

    sml = lambda i: (0, 0)
    return pl.pallas_call(
        body,
        grid=(_EPAD // RZ,),
        in_specs=[
            pl.BlockSpec((RZ, D), lambda i: (i, 0)),
            pl.BlockSpec((NCJ, _CH), lambda i: (i, 0)),
            pl.BlockSpec((NCJ, _CH), lambda i: (i, 0)),
            pl.BlockSpec((NCJ, _CH), lambda i: (i, 0)),
            pl.BlockSpec((EH, _CH), sml),
            pl.BlockSpec((EH, _CH), sml),
            pl.BlockSpec((EH, _CH), sml),
            pl.BlockSpec((EH, _CH), sml),
            pl.BlockSpec((EH, _CH), sml),
            pl.BlockSpec((1, 1), sml),
        ],
        out_specs=pl.BlockSpec((NCJ, _CH), lambda i: (i, 0)),
        out_shape=jax.ShapeDtypeStruct((_EPAD // _CH, _CH), jnp.float32),
    )(zp, angp, gtp, actp, wa, wg, wc, berow, worow, bo11)


def kernel(node_features, edge_index, angles, gt_edges, actions, round_n,
           W1, b1, W2, b2, We, be, Wo, bo):
    nf = node_features.astype(jnp.float32)
    src2 = jnp.concatenate(
        [edge_index[0], edge_index[1], jnp.zeros((_AGG_RT * _NW * _CH - E2,),
                                                 jnp.int32)])
    dst2 = jnp.concatenate(
        [edge_index[1], edge_index[0], jnp.zeros((_AGG_RT * _NW * _CH - E2,),
                                                 jnp.int32)])
    srcr = src2.reshape(_AGG_RT * _NW, _CH)
    dstr = dst2.reshape(_AGG_RT * _NW, _CH)
    epad = jnp.zeros((_RO_RT * _NW * _CH - E,), jnp.int32)
    srcr_e = jnp.concatenate([edge_index[0], epad]).reshape(_RO_RT * _NW, _CH)
    dstr_e = jnp.concatenate([edge_index[1], epad]).reshape(_RO_RT * _NW, _CH)
    rn = jnp.asarray(round_n, jnp.float32)

    zrows = jnp.zeros((_NPAD // _NS, _HD), jnp.float32)
    zdeg = jnp.zeros((_NPAD,), jnp.float32)

    X1 = _tc_matmul_bias(nf, W1[:D], W1[D:D + 1] * rn)
    p1, degn_raw = _sc_agg(X1, srcr, dstr, zrows, zdeg)
    degn = degn_raw[:N, None]
    X2 = _tc_norm_mm(p1, degn, b1[None, :], W2, relu=True, split_out=False)
    p2, _degn2 = _sc_agg(X2, srcr, dstr, zrows, zdeg)
    Wsd = jnp.concatenate([We[:D], We[D:2 * D]], axis=1)
    Ps, Pd = _tc_norm_mm(p2, degn, b2[None, :], Wsd, relu=False,
                         split_out=True)
    zp = _sc_readout_gather(Ps, Pd, srcr_e, dstr_e)
    fpad = jnp.zeros((_EPAD - E,), jnp.float32)
    nrow = _EPAD // _CH
    angp = jnp.concatenate([angles, fpad]).reshape(nrow, _CH)
    gtp = jnp.concatenate([gt_edges, fpad]).reshape(nrow, _CH)
    actp = jnp.concatenate([actions, fpad]).reshape(nrow, _CH)
    wcol = lambda v: jnp.tile(v[:, None], (1, _CH))
    qp = _tc_readout(zp, angp, gtp, actp,
                     wcol(We[2 * D]), wcol(We[2 * D + 1]),
                     wcol(We[2 * D + 2]), wcol(be), wcol(Wo[:, 0]),
                     bo[None, :])
    q = qp.reshape(-1)[:E]
    return q, q


# speedup vs baseline: 1.1719x; 1.1719x over previous
import dataclasses
import functools

import jax
import jax.numpy as jnp
from jax import lax
from jax.experimental import pallas as pl
from jax.experimental.pallas import tpu as pltpu
from jax.experimental.pallas import tpu_sc as plsc

N = 10000
E = 160000
E2 = 2 * E
D = 128
EH = 64

_NC = 2
_NS = 16
_NW = _NC * _NS
_CH = 128


def _vector_mesh():
    return plsc.VectorSubcoreMesh(core_axis_name="c", subcore_axis_name="s")


def _sc_params(tc_tiling=None):
    cp = pltpu.CompilerParams()
    fields = pltpu.CompilerParams.__dataclass_fields__
    if "needs_layout_passes" in fields:
        cp = dataclasses.replace(cp, needs_layout_passes=False)
    if tc_tiling is not None and "use_tc_tiling_on_sc" in fields:
        cp = dataclasses.replace(cp, use_tc_tiling_on_sc=tc_tiling)
    return cp


_NPAD = 10240
_HD = D // 2
_AGG_RT = 160
_AGG_NCH = E2 // _CH


def _sc_agg(x2, srcr, dstr, zrows, zdeg):
    nrow_t = _NPAD // _NS

    @functools.partial(
        pl.kernel,
        out_type=(
            jax.ShapeDtypeStruct((_NC, N, _HD), jnp.float32),
            jax.ShapeDtypeStruct((_NPAD,), jnp.float32),
        ),
        mesh=_vector_mesh(),
        compiler_params=_sc_params(tc_tiling=False),
        scratch_types=[
            pltpu.VMEM((_AGG_RT, _CH), jnp.int32),
            pltpu.VMEM((_AGG_RT, _CH), jnp.int32),
            pltpu.VMEM((_CH, _HD), jnp.float32),
            pltpu.VMEM((_CH, _HD), jnp.float32),
            pltpu.VMEM((_NPAD,), jnp.float32),
            pltpu.VMEM((nrow_t,), jnp.float32),
            pltpu.VMEM((nrow_t,), jnp.float32),
            pltpu.VMEM_SHARED((_NPAD, _HD), jnp.float32),
            pltpu.VMEM_SHARED((_NS, _NPAD), jnp.float32),
            pltpu.SemaphoreType.DMA,
            pltpu.SemaphoreType.DMA,
        ],
    )
    def k(x_hbm, src_hbm, dst_hbm, zr_hbm, zd_hbm, part_hbm, degn_hbm,
          idx_s, idx_d, rows0, rows1, deg_v, red0, red1, acc_sh, degst_sh,
          sem0, sem1):
        cid = lax.axis_index("c")
        sid = lax.axis_index("s")
        r0 = sid * nrow_t
        base = sid * _AGG_RT
        pltpu.sync_copy(zd_hbm, deg_v)
        pltpu.sync_copy(zr_hbm, acc_sh.at[pl.ds(r0, nrow_t)])
        pltpu.sync_copy(src_hbm.at[pl.ds(base, _AGG_RT)], idx_s)
        pltpu.sync_copy(dst_hbm.at[pl.ds(base, _AGG_RT)], idx_d)
        plsc.subcore_barrier()

        ones16 = jnp.full((16,), 1.0, jnp.float32)

        def hist(i):
            @pl.when(cid == 0)
            def _():
                @pl.loop(0, _CH, step=16)
                def _(j):
                    plsc.addupdate_scatter(deg_v, [idx_d[i, pl.ds(j, 16)]],
                                           ones16)

        def start_gather(i, buf, sem):
            pltpu.async_copy(x_hbm.at[cid].at[idx_s.at[i]], buf, sem)

        def wait_gather(buf, sem):
            pltpu.make_async_copy(x_hbm.at[cid].at[idx_s.at[0]], buf,
                                  sem).wait()

        def scatter(i, buf):
            pltpu.sync_copy(buf, acc_sh.at[idx_d.at[i]], add=True)

        start_gather(0, rows0, sem0)

        @pl.loop(0, _AGG_RT // 2)
        def _(t):
            i0 = 2 * t
            i1 = i0 + 1

            @pl.when(base + i1 < _AGG_NCH)
            def _():
                start_gather(i1, rows1, sem1)

            @pl.when(base + i0 < _AGG_NCH)
            def _():
                hist(i0)
                wait_gather(rows0, sem0)
                scatter(i0, rows0)

            @pl.when(jnp.logical_and(i0 + 2 < _AGG_RT,
                                     base + i0 + 2 < _AGG_NCH))
            def _():
                start_gather(i0 + 2, rows0, sem0)

            @pl.when(base + i1 < _AGG_NCH)
            def _():
                hist(i1)
                wait_gather(rows1, sem1)
                scatter(i1, rows1)

        @pl.when(cid == 0)
        def _():
            pltpu.sync_copy(deg_v, degst_sh.at[sid])

        plsc.subcore_barrier()

        @pl.when(cid == 0)
        def _():
            pltpu.sync_copy(degst_sh.at[0, pl.ds(r0, nrow_t)], red0)
            for t in range(1, _NS):
                pltpu.sync_copy(degst_sh.at[t, pl.ds(r0, nrow_t)], red1)

                @pl.loop(0, nrow_t, step=16)
                def _(j):
                    red0[pl.ds(j, 16)] = red0[pl.ds(j, 16)] + red1[pl.ds(j, 16)]

            pltpu.sync_copy(red0, degn_hbm.at[pl.ds(r0, nrow_t)])

        @pl.when(sid < _NS - 1)
        def _():
            pltpu.sync_copy(acc_sh.at[pl.ds(r0, nrow_t)],
                            part_hbm.at[cid, pl.ds(r0, nrow_t)])

        @pl.when(sid == _NS - 1)
        def _():
            pltpu.sync_copy(acc_sh.at[pl.ds(r0, N - (_NS - 1) * nrow_t)],
                            part_hbm.at[cid, pl.ds(r0, N - (_NS - 1) * nrow_t)])

    return k(x2, srcr, dstr, zrows, zdeg)


_RO_RT = 40
_RO_NCH = E // _CH
_EPAD = _RO_RT * _NW * _CH


def _sc_readout_gather(ps, pd, srcr, dstr):
    @functools.partial(
        pl.kernel,
        out_type=jax.ShapeDtypeStruct((_EPAD, EH), jnp.float32),
        mesh=_vector_mesh(),
        compiler_params=_sc_params(tc_tiling=False),
        scratch_types=[
            pltpu.VMEM((_RO_RT, _CH), jnp.int32),
            pltpu.VMEM((_RO_RT, _CH), jnp.int32),
            pltpu.VMEM((_CH, EH), jnp.float32),
            pltpu.VMEM((_CH, EH), jnp.float32),
            pltpu.VMEM((_CH, EH), jnp.float32),
            pltpu.VMEM((_CH, EH), jnp.float32),
            pltpu.SemaphoreType.DMA,
            pltpu.SemaphoreType.DMA,
            pltpu.SemaphoreType.DMA,
            pltpu.SemaphoreType.DMA,
            pltpu.SemaphoreType.DMA,
            pltpu.SemaphoreType.DMA,
        ],
    )
    def k(ps_hbm, pd_hbm, s_hbm, d_hbm, z_hbm, idx_s, idx_d,
          rsA, rdA, rsB, rdB, gsA, gdA, goA, gsB, gdB, goB):
        cid = lax.axis_index("c")
        sid = lax.axis_index("s")
        wid = cid * _NS + sid
        base = wid * _RO_RT
        pltpu.sync_copy(s_hbm.at[pl.ds(base, _RO_RT)], idx_s)
        pltpu.sync_copy(d_hbm.at[pl.ds(base, _RO_RT)], idx_d)

        def start_gathers(i, rs, rd, ss, sd):
            pltpu.async_copy(ps_hbm.at[idx_s.at[i]], rs, ss)
            pltpu.async_copy(pd_hbm.at[idx_d.at[i]], rd, sd)

        def wait_gathers(rs, rd, ss, sd):
            pltpu.make_async_copy(ps_hbm.at[idx_s.at[0]], rs, ss).wait()
            pltpu.make_async_copy(pd_hbm.at[idx_d.at[0]], rd, sd).wait()

        def add_rows(rs, rd):
            @pl.loop(0, _CH)
            def _(r):
                for j in range(EH // 16):
                    sl = pl.ds(j * 16, 16)
                    rs[r, sl] = rs[r, sl] + rd[r, sl]

        def start_store(i, rs, so):
            pltpu.async_copy(rs, z_hbm.at[pl.ds((base + i) * _CH, _CH)], so)

        def drain_store(rs, so):
            pltpu.make_async_copy(rs, z_hbm.at[pl.ds(0, _CH)], so).wait()

        start_gathers(0, rsA, rdA, gsA, gdA)

        @pl.loop(0, _RO_RT // 2)
        def _(t):
            i0 = 2 * t
            i1 = i0 + 1
            i2 = i0 + 2

            @pl.when(base + i1 < _RO_NCH)
            def _():
                @pl.when(t > 0)
                def _():
                    drain_store(rsB, goB)

                start_gathers(i1, rsB, rdB, gsB, gdB)

            @pl.when(base + i0 < _RO_NCH)
            def _():
                wait_gathers(rsA, rdA, gsA, gdA)
                add_rows(rsA, rdA)
                start_store(i0, rsA, goA)

            @pl.when(jnp.logical_and(i2 < _RO_RT, base + i2 < _RO_NCH))
            def _():
                drain_store(rsA, goA)
                start_gathers(i2, rsA, rdA, gsA, gdA)

            @pl.when(base + i1 < _RO_NCH)
            def _():
                wait_gathers(rsB, rdB, gsB, gdB)
                add_rows(rsB, rdB)
                start_store(i1, rsB, goB)

        @pl.when(base < _RO_NCH)
        def _():
            drain_store(rsA, goA)

        @pl.when(base + 1 < _RO_NCH)
        def _():
            drain_store(rsB, goB)

    return k(ps, pd, srcr, dstr)


def _tc_matmul_bias(x, w, brow):
    R = 2000

    def body(x_ref, w_ref, b_ref, o_ref):
        y = jnp.dot(x_ref[...], w_ref[...],
                    preferred_element_type=jnp.float32) + b_ref[...]
        o_ref[0] = y[:, :_HD]
        o_ref[1] = y[:, _HD:]

    return pl.pallas_call(
        body,
        grid=(N // R,),
        in_specs=[
            pl.BlockSpec((R, D), lambda i: (i, 0)),
            pl.BlockSpec((D, D), lambda i: (0, 0)),
            pl.BlockSpec((1, D), lambda i: (0, 0)),
        ],
        out_specs=pl.BlockSpec((2, R, _HD), lambda i: (0, i, 0)),
        out_shape=jax.ShapeDtypeStruct((2, N, _HD), jnp.float32),
    )(x, w, brow)


def _tc_norm_mm(part, degn, brow, w, relu, split_out):
    R = 2000

    def body(p_ref, d_ref, b_ref, w_ref, *o_refs):
        agg = jnp.concatenate([p_ref[0], p_ref[1]], axis=1)
        deg = jnp.maximum(d_ref[...], 1.0)
        h = agg / deg + b_ref[...]
        if relu:
            h = jnp.maximum(h, 0.0)
        ya = jnp.dot(h, w_ref[:, :_HD], preferred_element_type=jnp.float32)
        yb = jnp.dot(h, w_ref[:, _HD:], preferred_element_type=jnp.float32)
        if split_out:
            o_refs[0][...] = ya
            o_refs[1][...] = yb
        else:
            o_refs[0][0] = ya
            o_refs[0][1] = yb

    if split_out:
        out_specs = [pl.BlockSpec((R, _HD), lambda i: (i, 0)),
                     pl.BlockSpec((R, _HD), lambda i: (i, 0))]
        out_shape = [jax.ShapeDtypeStruct((N, _HD), jnp.float32),
                     jax.ShapeDtypeStruct((N, _HD), jnp.float32)]
    else:
        out_specs = pl.BlockSpec((2, R, _HD), lambda i: (0, i, 0))
        out_shape = jax.ShapeDtypeStruct((2, N, _HD), jnp.float32)

    return pl.pallas_call(
        body,
        grid=(N // R,),
        in_specs=[
            pl.BlockSpec((2, R, _HD), lambda i: (0, i, 0)),
            pl.BlockSpec((R, 1), lambda i: (i, 0)),
            pl.BlockSpec((1, D), lambda i: (0, 0)),
            pl.BlockSpec((D, D), lambda i: (0, 0)),
        ],
        out_specs=out_specs,
        out_shape=out_shape,
    )(part, degn, brow, w)


def _tc_readout(zp, angp, gtp, actp, wa, wg, wc, berow, worow, bo11):
    NCJ = 64
    RH = NCJ * EH

    def body(z_ref, a_ref, g_ref, c_ref, wa_ref, wg_ref, wc_ref, be_ref,
             wo_ref, bo_ref, o_ref):
        wa = wa_ref[...]
        wg = wg_ref[...]
        wc = wc_ref[...]
        be = be_ref[...]
        wo = wo_ref[...]
        bo = bo_ref[0, 0]
        for j in range(NCJ):
            zt = jnp.transpose(z_ref[pl.ds(j * EH, EH), :])
            for h in range(2):
                r = 2 * j + h
                zz = (zt[h * EH:(h + 1) * EH]
                      + a_ref[r:r + 1, :] * wa + g_ref[r:r + 1, :] * wg
                      + c_ref[r:r + 1, :] * wc + be)
                hh = jnp.maximum(zz, 0.0)
                o_ref[r:r + 1, :] = (
                    jnp.sum(hh * wo, axis=0, keepdims=True) + bo)

    sml = lambda i: (0, 0)
    nrow = _EPAD // EH
    return pl.pallas_call(
        body,
        grid=(_EPAD // (NCJ * _CH),),
        in_specs=[
            pl.BlockSpec((RH, D), lambda i: (i, 0)),
            pl.BlockSpec((2 * NCJ, EH), lambda i: (i, 0)),
            pl.BlockSpec((2 * NCJ, EH), lambda i: (i, 0)),
            pl.BlockSpec((2 * NCJ, EH), lambda i: (i, 0)),
            pl.BlockSpec((EH, EH), sml),
            pl.BlockSpec((EH, EH), sml),
            pl.BlockSpec((EH, EH), sml),
            pl.BlockSpec((EH, EH), sml),
            pl.BlockSpec((EH, EH), sml),
            pl.BlockSpec((1, 1), sml),
        ],
        out_specs=pl.BlockSpec((2 * NCJ, EH), lambda i: (i, 0)),
        out_shape=jax.ShapeDtypeStruct((nrow, EH), jnp.float32),
    )(zp, angp, gtp, actp, wa, wg, wc, berow, worow, bo11)


def kernel(node_features, edge_index, angles, gt_edges, actions, round_n,
           W1, b1, W2, b2, We, be, Wo, bo):
    nf = node_features.astype(jnp.float32)
    src2 = jnp.concatenate(
        [edge_index[0], edge_index[1], jnp.zeros((_AGG_RT * _NW * _CH - E2,),
                                                 jnp.int32)])
    dst2 = jnp.concatenate(
        [edge_index[1], edge_index[0], jnp.zeros((_AGG_RT * _NW * _CH - E2,),
                                                 jnp.int32)])
    srcr = src2.reshape(_AGG_RT * _NW, _CH)
    dstr = dst2.reshape(_AGG_RT * _NW, _CH)
    epad = jnp.zeros((_RO_RT * _NW * _CH - E,), jnp.int32)

    def _shuf(v):
        r = jnp.concatenate([v, epad]).reshape(_RO_RT * _NW, 2, EH)
        return r.transpose(0, 2, 1).reshape(_RO_RT * _NW, _CH)

    srcr_e = _shuf(edge_index[0])
    dstr_e = _shuf(edge_index[1])
    rn = jnp.asarray(round_n, jnp.float32)

    zrows = jnp.zeros((_NPAD // _NS, _HD), jnp.float32)
    zdeg = jnp.zeros((_NPAD,), jnp.float32)

    X1 = _tc_matmul_bias(nf, W1[:D], W1[D:D + 1] * rn)
    p1, degn_raw = _sc_agg(X1, srcr, dstr, zrows, zdeg)
    degn = degn_raw[:N, None]
    X2 = _tc_norm_mm(p1, degn, b1[None, :], W2, relu=True, split_out=False)
    p2, _degn2 = _sc_agg(X2, srcr, dstr, zrows, zdeg)
    Wsd = jnp.concatenate([We[:D], We[D:2 * D]], axis=1)
    Ps, Pd = _tc_norm_mm(p2, degn, b2[None, :], Wsd, relu=False,
                         split_out=True)
    zp = _sc_readout_gather(Ps, Pd, srcr_e, dstr_e)
    z2 = zp.reshape(_EPAD // 2, D)
    fpad = jnp.zeros((_EPAD - E,), jnp.float32)
    nrow = _EPAD // EH
    angp = jnp.concatenate([angles, fpad]).reshape(nrow, EH)
    gtp = jnp.concatenate([gt_edges, fpad]).reshape(nrow, EH)
    actp = jnp.concatenate([actions, fpad]).reshape(nrow, EH)
    wcol = lambda v: jnp.tile(v[:, None], (1, EH))
    qp = _tc_readout(z2, angp, gtp, actp,
                     wcol(We[2 * D]), wcol(We[2 * D + 1]),
                     wcol(We[2 * D + 2]), wcol(be), wcol(Wo[:, 0]),
                     bo[None, :])
    q = qp.reshape(-1)[:E]
    return q, q
